# baseline (speedup 1.0000x reference)
import functools

import jax
import jax.numpy as jnp
from jax import lax
from jax.experimental import pallas as pl
from jax.experimental.pallas import tpu as pltpu
from jax.experimental.pallas import tpu_sc as plsc

_B = 16384
_D = 16
_NC = 2
_NS = 16
_NW = _NC * _NS
_BPW = _B // _NW
_CH = 128
_NCH = _BPW // _CH


def _gather_body(user_tbl, item_tbl, uid2d, iid2d, ue_out, ie_out,
                 uidx, iidx, urows, irows, sem):
    wid = lax.axis_index("s") * _NC + lax.axis_index("c")
    pltpu.sync_copy(uid2d.at[pl.ds(wid * _NCH, _NCH)], uidx)
    pltpu.sync_copy(iid2d.at[pl.ds(wid * _NCH, _NCH)], iidx)
    copies = []
    for j in range(_NCH):
        copies.append(
            pltpu.async_copy(user_tbl.at[uidx.at[j]], urows.at[j], sem))
        copies.append(
            pltpu.async_copy(item_tbl.at[iidx.at[j]], irows.at[j], sem))
    for c in copies:
        c.wait()
    pltpu.sync_copy(urows, ue_out.at[pl.ds(wid * _NCH, _NCH)])
    pltpu.sync_copy(irows, ie_out.at[pl.ds(wid * _NCH, _NCH)])


@jax.jit
def _sc_gather(user_table, item_table, uid2d, iid2d):
    mesh = plsc.VectorSubcoreMesh(core_axis_name="c", subcore_axis_name="s")
    out3 = (_NW * _NCH, _CH, _D)
    f = pl.kernel(
        _gather_body,
        out_type=[
            jax.ShapeDtypeStruct(out3, jnp.float32),
            jax.ShapeDtypeStruct(out3, jnp.float32),
        ],
        mesh=mesh,
        scratch_types=[
            pltpu.VMEM((_NCH, _CH), jnp.int32),
            pltpu.VMEM((_NCH, _CH), jnp.int32),
            pltpu.VMEM((_NCH, _CH, _D), jnp.float32),
            pltpu.VMEM((_NCH, _CH, _D), jnp.float32),
            pltpu.SemaphoreType.DMA,
        ],
        compiler_params=pltpu.CompilerParams(use_tc_tiling_on_sc=False),
    )
    return f(user_table, item_table, uid2d, iid2d)


def _mlp_body(ue_ref, ie_ref, w1u_ref, w1i_ref, b1_ref, gamma_ref, beta_ref,
              w2t_ref, b2_ref, out_ref):
    h = jnp.dot(ue_ref[...], w1u_ref[...], preferred_element_type=jnp.float32)
    h = h + jnp.dot(ie_ref[...], w1i_ref[...],
                    preferred_element_type=jnp.float32)
    h = h + b1_ref[...]
    h = jnp.maximum(h, 0.0)
    mean = jnp.mean(h, axis=0, keepdims=True)
    c = h - mean
    var = jnp.mean(c * c, axis=0, keepdims=True)
    hn = c * lax.rsqrt(var + 1e-5) * gamma_ref[...] + beta_ref[...]
    logit = jnp.sum(hn * w2t_ref[...], axis=1) + b2_ref[0]
    out_ref[...] = 1.0 / (1.0 + jnp.exp(-logit))


@functools.partial(jax.jit, static_argnames=("interpret",))
def _tc_mlp(ue, ie, W1, b1, gamma, beta, W2, b2, interpret=False):
    w1u = W1[:_D, :]
    w1i = W1[_D:, :]
    b1r = b1.reshape(1, _D)
    gr = gamma.reshape(1, _D)
    br = beta.reshape(1, _D)
    w2r = W2.reshape(1, _D)
    b2r = b2.reshape(1)
    return pl.pallas_call(
        _mlp_body,
        out_shape=jax.ShapeDtypeStruct((_B,), jnp.float32),
        in_specs=[
            pl.BlockSpec(memory_space=pltpu.VMEM),
            pl.BlockSpec(memory_space=pltpu.VMEM),
            pl.BlockSpec(memory_space=pltpu.VMEM),
            pl.BlockSpec(memory_space=pltpu.VMEM),
            pl.BlockSpec(memory_space=pltpu.VMEM),
            pl.BlockSpec(memory_space=pltpu.VMEM),
            pl.BlockSpec(memory_space=pltpu.VMEM),
            pl.BlockSpec(memory_space=pltpu.VMEM),
            pl.BlockSpec(memory_space=pltpu.SMEM),
        ],
        out_specs=pl.BlockSpec(memory_space=pltpu.VMEM),
        interpret=interpret,
    )(ue, ie, w1u, w1i, b1r, gr, br, w2r, b2r)


def kernel(user_id, item_id, user_table, item_table, W1, b1, gamma, beta,
           W2, b2):
    uid2d = user_id.reshape(_NW * _NCH, _CH)
    iid2d = item_id.reshape(_NW * _NCH, _CH)
    ue3, ie3 = _sc_gather(user_table, item_table, uid2d, iid2d)
    ue = ue3.reshape(_B, _D)
    ie = ie3.reshape(_B, _D)
    y = _tc_mlp(ue, ie, W1, b1, gamma, beta, W2, b2)
    return y.reshape(_B, 1)
